# baseline (speedup 1.0000x reference)
import jax
import jax.numpy as jnp
from jax.experimental import pallas as pl
from jax.experimental.pallas import tpu as pltpu

T, N, D, H = 512, 16, 512, 512
C = 8
NB = T // C


def _gates(pre, c_masked):
    i_g = jax.nn.sigmoid(pre[:, 0 * H:1 * H])
    f_g = jax.nn.sigmoid(pre[:, 1 * H:2 * H])
    g_g = jnp.tanh(pre[:, 2 * H:3 * H])
    o_g = jax.nn.sigmoid(pre[:, 3 * H:4 * H])
    c_new = f_g * c_masked + i_g * g_g
    h_new = o_g * jnp.tanh(c_new)
    return h_new, c_new


def _lstm2_kernel(x_ref, m0_ref, m1_ref,
                  wih0_ref, whh0_ref, b0_ref,
                  wih1_ref, whh1_ref, b1_ref,
                  h0i_ref, c0i_ref, h1i_ref, c1i_ref,
                  ys_ref, hT0_ref, cT0_ref, hT1_ref, cT1_ref,
                  g0_s, g1_s, y0_s, h0_s, c0_s, h1_s, c1_s):
    i = pl.program_id(0)

    @pl.when(i == 0)
    def _init0():
        h0_s[:] = h0i_ref[:]
        c0_s[:] = c0i_ref[:]

    @pl.when(i == 1)
    def _init1():
        h1_s[:] = h1i_ref[:]
        c1_s[:] = c1i_ref[:]

    g1_s[:] = (jnp.dot(y0_s[:], wih1_ref[:],
                       preferred_element_type=jnp.float32) + b1_ref[:])
    g0_s[:] = (jnp.dot(x_ref[:], wih0_ref[:],
                       preferred_element_type=jnp.float32) + b0_ref[:])

    def step(j):
        r = pl.ds(j * N, N)
        m0 = m0_ref[r, :]
        m1 = m1_ref[r, :]
        hm0 = (h0_s[:] * m0).astype(jnp.bfloat16)
        cm0 = c0_s[:] * m0
        hm1 = (h1_s[:] * m1).astype(jnp.bfloat16)
        cm1 = c1_s[:] * m1
        pre0 = g0_s[r, :] + jnp.dot(hm0, whh0_ref[:],
                                    preferred_element_type=jnp.float32)
        pre1 = g1_s[r, :] + jnp.dot(hm1, whh1_ref[:],
                                    preferred_element_type=jnp.float32)
        h0n, c0n = _gates(pre0, cm0)
        h1n, c1n = _gates(pre1, cm1)
        h0_s[:] = h0n
        c0_s[:] = c0n
        h1_s[:] = h1n
        c1_s[:] = c1n
        y0_s[r, :] = h0n.astype(jnp.bfloat16)
        ys_ref[r, :] = h1n

    for j in range(C):
        step(j)

    @pl.when(i == NB - 1)
    def _fin0():
        hT0_ref[:] = h0_s[:]
        cT0_ref[:] = c0_s[:]

    @pl.when(i == NB)
    def _fin1():
        hT1_ref[:] = h1_s[:]
        cT1_ref[:] = c1_s[:]


def kernel(x, rnn_states, dones, W_ih0, W_hh0, b_ih0, b_hh0,
           W_ih1, W_hh1, b_ih1, b_hh1):
    masks = (1 - dones).astype(jnp.float32).reshape(T * N, 1)
    b0 = (b_ih0 + b_hh0).reshape(1, 4 * H)
    b1 = (b_ih1 + b_hh1).reshape(1, 4 * H)

    full = lambda shape: pl.BlockSpec(shape, lambda i: (0,) * len(shape))
    ys, hT0, cT0, hT1, cT1 = pl.pallas_call(
        _lstm2_kernel,
        grid=(NB + 1,),
        in_specs=[
            pl.BlockSpec((C * N, D), lambda i: (jnp.minimum(i, NB - 1), 0)),
            pl.BlockSpec((C * N, 1), lambda i: (jnp.minimum(i, NB - 1), 0)),
            pl.BlockSpec((C * N, 1), lambda i: (jnp.maximum(i - 1, 0), 0)),
            full((D, 4 * H)),
            full((H, 4 * H)),
            full((1, 4 * H)),
            full((H, 4 * H)),
            full((H, 4 * H)),
            full((1, 4 * H)),
            full((N, H)),
            full((N, H)),
            full((N, H)),
            full((N, H)),
        ],
        out_specs=[
            pl.BlockSpec((C * N, H), lambda i: (jnp.maximum(i - 1, 0), 0)),
            full((N, H)),
            full((N, H)),
            full((N, H)),
            full((N, H)),
        ],
        out_shape=[
            jax.ShapeDtypeStruct((T * N, H), jnp.float32),
            jax.ShapeDtypeStruct((N, H), jnp.float32),
            jax.ShapeDtypeStruct((N, H), jnp.float32),
            jax.ShapeDtypeStruct((N, H), jnp.float32),
            jax.ShapeDtypeStruct((N, H), jnp.float32),
        ],
        scratch_shapes=[
            pltpu.VMEM((C * N, 4 * H), jnp.float32),
            pltpu.VMEM((C * N, 4 * H), jnp.float32),
            pltpu.VMEM((C * N, H), jnp.bfloat16),
            pltpu.VMEM((N, H), jnp.float32),
            pltpu.VMEM((N, H), jnp.float32),
            pltpu.VMEM((N, H), jnp.float32),
            pltpu.VMEM((N, H), jnp.float32),
        ],
    )(x.astype(jnp.bfloat16), masks, masks,
      W_ih0.T.astype(jnp.bfloat16), W_hh0.T.astype(jnp.bfloat16), b0,
      W_ih1.T.astype(jnp.bfloat16), W_hh1.T.astype(jnp.bfloat16), b1,
      rnn_states[0], rnn_states[2], rnn_states[1], rnn_states[3])
    final = jnp.stack([hT0, hT1, cT0, cT1], axis=0)
    return ys, final


# speedup vs baseline: 1.0217x; 1.0217x over previous
import jax
import jax.numpy as jnp
from jax.experimental import pallas as pl
from jax.experimental.pallas import tpu as pltpu

T, N, D, H = 512, 16, 512, 512
C = 16
NB = T // C


def _gates(pre, c_masked):
    i_g = jax.nn.sigmoid(pre[:, 0 * H:1 * H])
    f_g = jax.nn.sigmoid(pre[:, 1 * H:2 * H])
    g_g = jnp.tanh(pre[:, 2 * H:3 * H])
    o_g = jax.nn.sigmoid(pre[:, 3 * H:4 * H])
    c_new = f_g * c_masked + i_g * g_g
    h_new = o_g * jnp.tanh(c_new)
    return h_new, c_new


def _lstm2_kernel(x_ref, m0_ref, m1_ref,
                  wih0_ref, whh0_ref, b0_ref,
                  wih1_ref, whh1_ref, b1_ref,
                  h0i_ref, c0i_ref, h1i_ref, c1i_ref,
                  ys_ref, hT0_ref, cT0_ref, hT1_ref, cT1_ref,
                  g0_s, g1_s, y0_s, h0_s, c0_s, h1_s, c1_s):
    i = pl.program_id(0)

    @pl.when(i == 0)
    def _init0():
        h0_s[:] = h0i_ref[:]
        c0_s[:] = c0i_ref[:]

    @pl.when(i == 1)
    def _init1():
        h1_s[:] = h1i_ref[:]
        c1_s[:] = c1i_ref[:]

    g1_s[:] = (jax.lax.dot_general(
        y0_s[:], wih1_ref[:], (((1,), (1,)), ((), ())),
        preferred_element_type=jnp.float32) + b1_ref[:])
    g0_s[:] = (jax.lax.dot_general(
        x_ref[:], wih0_ref[:], (((1,), (1,)), ((), ())),
        preferred_element_type=jnp.float32) + b0_ref[:])

    def step(j):
        r = pl.ds(j * N, N)
        m0 = m0_ref[r, :]
        m1 = m1_ref[r, :]
        hm0 = (h0_s[:] * m0).astype(jnp.bfloat16)
        cm0 = c0_s[:] * m0
        hm1 = (h1_s[:] * m1).astype(jnp.bfloat16)
        cm1 = c1_s[:] * m1
        pre0 = g0_s[r, :] + jnp.dot(hm0, whh0_ref[:],
                                    preferred_element_type=jnp.float32)
        pre1 = g1_s[r, :] + jnp.dot(hm1, whh1_ref[:],
                                    preferred_element_type=jnp.float32)
        h0n, c0n = _gates(pre0, cm0)
        h1n, c1n = _gates(pre1, cm1)
        h0_s[:] = h0n
        c0_s[:] = c0n
        h1_s[:] = h1n
        c1_s[:] = c1n
        y0_s[r, :] = h0n.astype(jnp.bfloat16)
        ys_ref[r, :] = h1n

    for j in range(C):
        step(j)

    @pl.when(i == NB - 1)
    def _fin0():
        hT0_ref[:] = h0_s[:]
        cT0_ref[:] = c0_s[:]

    @pl.when(i == NB)
    def _fin1():
        hT1_ref[:] = h1_s[:]
        cT1_ref[:] = c1_s[:]


def kernel(x, rnn_states, dones, W_ih0, W_hh0, b_ih0, b_hh0,
           W_ih1, W_hh1, b_ih1, b_hh1):
    masks = (1 - dones).astype(jnp.bfloat16).reshape(T * N, 1)
    b0 = (b_ih0 + b_hh0).reshape(1, 4 * H)
    b1 = (b_ih1 + b_hh1).reshape(1, 4 * H)

    full = lambda shape: pl.BlockSpec(shape, lambda i: (0,) * len(shape))
    ys, hT0, cT0, hT1, cT1 = pl.pallas_call(
        _lstm2_kernel,
        grid=(NB + 1,),
        in_specs=[
            pl.BlockSpec((C * N, D), lambda i: (jnp.minimum(i, NB - 1), 0)),
            pl.BlockSpec((C * N, 1), lambda i: (jnp.minimum(i, NB - 1), 0)),
            pl.BlockSpec((C * N, 1), lambda i: (jnp.maximum(i - 1, 0), 0)),
            full((4 * H, D)),
            full((H, 4 * H)),
            full((1, 4 * H)),
            full((4 * H, H)),
            full((H, 4 * H)),
            full((1, 4 * H)),
            full((N, H)),
            full((N, H)),
            full((N, H)),
            full((N, H)),
        ],
        out_specs=[
            pl.BlockSpec((C * N, H), lambda i: (jnp.maximum(i - 1, 0), 0)),
            full((N, H)),
            full((N, H)),
            full((N, H)),
            full((N, H)),
        ],
        out_shape=[
            jax.ShapeDtypeStruct((T * N, H), jnp.float32),
            jax.ShapeDtypeStruct((N, H), jnp.float32),
            jax.ShapeDtypeStruct((N, H), jnp.float32),
            jax.ShapeDtypeStruct((N, H), jnp.float32),
            jax.ShapeDtypeStruct((N, H), jnp.float32),
        ],
        scratch_shapes=[
            pltpu.VMEM((C * N, 4 * H), jnp.float32),
            pltpu.VMEM((C * N, 4 * H), jnp.float32),
            pltpu.VMEM((C * N, H), jnp.bfloat16),
            pltpu.VMEM((N, H), jnp.float32),
            pltpu.VMEM((N, H), jnp.float32),
            pltpu.VMEM((N, H), jnp.float32),
            pltpu.VMEM((N, H), jnp.float32),
        ],
    )(x.astype(jnp.bfloat16), masks, masks,
      W_ih0.astype(jnp.bfloat16), W_hh0.T.astype(jnp.bfloat16), b0,
      W_ih1.astype(jnp.bfloat16), W_hh1.T.astype(jnp.bfloat16), b1,
      rnn_states[0], rnn_states[2], rnn_states[1], rnn_states[3])
    final = jnp.stack([hT0, hT1, cT0, cT1], axis=0)
    return ys, final
